# baseline (speedup 1.0000x reference)
import functools

import jax
import jax.numpy as jnp
from jax import lax
from jax.experimental import pallas as pl
from jax.experimental.pallas import tpu as pltpu
from jax.experimental.pallas import tpu_sc as plsc

D = 64
KH = 3
NF = 32
HOUT = D - KH + 1
NOUT = HOUT * NF
CH = 128
LANES = 16
REPACK_BR = 2000


def _build_band_weights(conv_kernel):
    k = conv_kernel[:, :, 0, :]
    w = jnp.zeros((KH, D, HOUT, NF), jnp.float32)
    ii = jnp.arange(HOUT)
    for dh in range(KH):
        w = w.at[:, ii + dh, ii, :].set(k[dh][:, None, :])
    return w.reshape(KH, D, NOUT)


def _repack_body(lo_ref, hi_ref, o_ref):
    o_ref[...] = jnp.concatenate([lo_ref[...], hi_ref[...]], axis=-1)


def _repack_tc(table):
    n = table.shape[0]
    half_blocks = (n // 2) // REPACK_BR
    return pl.pallas_call(
        _repack_body,
        grid=(half_blocks,),
        in_specs=[pl.BlockSpec((REPACK_BR, D), lambda i: (i, 0)),
                  pl.BlockSpec((REPACK_BR, D),
                               lambda i: (i + half_blocks, 0))],
        out_specs=pl.BlockSpec((REPACK_BR, 2 * D), lambda i: (i, 0)),
        out_shape=jax.ShapeDtypeStruct((n // 2, 2 * D), jnp.float32),
    )(table, table)


def _conv_body(h2_ref, r2_ref, t2_ref, ph_ref, pr_ref, pt_ref,
               wh_ref, wr_ref, wt_ref, b_ref, o_ref):
    def sel(x2_ref, p_ref):
        return jnp.where(p_ref[...] > 0.5, x2_ref[:, D:], x2_ref[:, :D])
    acc = jnp.dot(sel(h2_ref, ph_ref), wh_ref[...],
                  preferred_element_type=jnp.float32)
    acc = acc + jnp.dot(sel(r2_ref, pr_ref), wr_ref[...],
                        preferred_element_type=jnp.float32)
    acc = acc + jnp.dot(sel(t2_ref, pt_ref), wt_ref[...],
                        preferred_element_type=jnp.float32)
    o_ref[...] = acc + b_ref[...]


def _conv_tc(h2, r2, t2, ph, pr, pt, wh, wr, wt, bias_row, block_b):
    b = h2.shape[0]
    grid = (b // block_b,)
    pair_spec = pl.BlockSpec((block_b, 2 * D), lambda i: (i, 0))
    par_spec = pl.BlockSpec((block_b, 1), lambda i: (i, 0))
    w_spec = pl.BlockSpec((D, NOUT), lambda i: (0, 0))
    return pl.pallas_call(
        _conv_body,
        grid=grid,
        in_specs=[pair_spec, pair_spec, pair_spec,
                  par_spec, par_spec, par_spec,
                  w_spec, w_spec, w_spec,
                  pl.BlockSpec((1, NOUT), lambda i: (0, 0))],
        out_specs=pl.BlockSpec((block_b, NOUT), lambda i: (i, 0)),
        out_shape=jax.ShapeDtypeStruct((b, NOUT), jnp.float32),
    )(h2, r2, t2, ph, pr, pt, wh, wr, wt, bias_row)


def _gather_sc(h_idx, r_idx, t_idx, ent2, rel2):
    b = h_idx.shape[0]
    e_half = ent2.shape[0]
    r_half = rel2.shape[0]
    info = plsc.get_sparse_core_info()
    nc = info.num_cores
    nw = nc * info.num_subcores
    rows_w = b // nw
    n_ch = rows_w // CH

    @functools.partial(
        pl.kernel,
        mesh=plsc.VectorSubcoreMesh(core_axis_name="c", subcore_axis_name="s"),
        compiler_params=pltpu.CompilerParams(use_tc_tiling_on_sc=True),
        out_type=(
            jax.ShapeDtypeStruct((b, 2 * D), jnp.float32),
            jax.ShapeDtypeStruct((b, 2 * D), jnp.float32),
            jax.ShapeDtypeStruct((b, 2 * D), jnp.float32),
        ),
        scratch_types=[
            pltpu.VMEM((rows_w,), jnp.int32),
            pltpu.VMEM((rows_w,), jnp.int32),
            pltpu.VMEM((rows_w,), jnp.int32),
            pltpu.VMEM((2, CH), jnp.int32),
            pltpu.VMEM((2, CH, 2 * D), jnp.float32),
            pltpu.SemaphoreType.DMA,
        ],
    )
    def gather_kernel(hi_hbm, ri_hbm, ti_hbm, ent_hbm, rel_hbm,
                      ho_hbm, ro_hbm, to_hbm,
                      hv, rv, tv, tidx, stag, sem):
        wid = lax.axis_index("s") * nc + lax.axis_index("c")
        base = wid * rows_w
        pltpu.sync_copy(hi_hbm.at[pl.ds(base, rows_w)], hv)
        pltpu.sync_copy(ri_hbm.at[pl.ds(base, rows_w)], rv)
        pltpu.sync_copy(ti_hbm.at[pl.ds(base, rows_w)], tv)

        def fill_tidx(idx_v, c, d, half):
            for k in range(CH // LANES):
                v = idx_v[pl.ds(c * CH + k * LANES, LANES)]
                v = v - jnp.where(v >= half, half, 0)
                tidx[d, pl.ds(k * LANES, LANES)] = v

        def start_gather(tab_hbm, d):
            return pltpu.async_copy(tab_hbm.at[tidx.at[d]], stag.at[d], sem)

        for idx_v, tab_hbm, out_hbm, half in ((hv, ent_hbm, ho_hbm, e_half),
                                              (rv, rel_hbm, ro_hbm, r_half),
                                              (tv, ent_hbm, to_hbm, e_half)):
            fill_tidx(idx_v, 0, 0, half)
            copies = {0: start_gather(tab_hbm, 0)}
            for c in range(n_ch):
                if c + 1 < n_ch:
                    fill_tidx(idx_v, c + 1, (c + 1) % 2, half)
                    copies[(c + 1) % 2] = start_gather(tab_hbm, (c + 1) % 2)
                copies[c % 2].wait()
                pltpu.sync_copy(stag.at[c % 2],
                                out_hbm.at[pl.ds(base + c * CH, CH)])

    return gather_kernel(h_idx, r_idx, t_idx, ent2, rel2)


def kernel(inputs, entity_embeddings, relation_embeddings, conv_kernel, conv_bias):
    b = inputs.shape[0]
    e_half = entity_embeddings.shape[0] // 2
    r_half = relation_embeddings.shape[0] // 2
    idx = inputs.astype(jnp.int32)
    ent2 = _repack_tc(entity_embeddings)
    rel2 = _repack_tc(relation_embeddings)
    h2, r2, t2 = _gather_sc(idx[:, 0], idx[:, 1], idx[:, 2], ent2, rel2)
    ph = (idx[:, 0] >= e_half).astype(jnp.float32)[:, None]
    pr = (idx[:, 1] >= r_half).astype(jnp.float32)[:, None]
    pt = (idx[:, 2] >= e_half).astype(jnp.float32)[:, None]
    w = _build_band_weights(conv_kernel)
    bias_row = jnp.tile(conv_bias, HOUT)[None, :]
    return _conv_tc(h2, r2, t2, ph, pr, pt, w[0], w[1], w[2], bias_row, 512)


# speedup vs baseline: 1.1344x; 1.1344x over previous
import functools

import jax
import jax.numpy as jnp
from jax import lax
from jax.experimental import pallas as pl
from jax.experimental.pallas import tpu as pltpu
from jax.experimental.pallas import tpu_sc as plsc

D = 64
KH = 3
NF = 32
HOUT = D - KH + 1
NOUT = HOUT * NF
CH = 128


def _build_band_weights(conv_kernel):
    k = conv_kernel[:, :, 0, :]
    w = jnp.zeros((KH, D, HOUT, NF), jnp.float32)
    ii = jnp.arange(HOUT)
    for dh in range(KH):
        w = w.at[:, ii + dh, ii, :].set(k[dh][:, None, :])
    return w.reshape(KH, D, NOUT)


def _conv_body(h_ref, r_ref, t_ref, wh_ref, wr_ref, wt_ref, b_ref, o_ref):
    acc = jnp.dot(h_ref[...], wh_ref[...], preferred_element_type=jnp.float32)
    acc = acc + jnp.dot(r_ref[...], wr_ref[...], preferred_element_type=jnp.float32)
    acc = acc + jnp.dot(t_ref[...], wt_ref[...], preferred_element_type=jnp.float32)
    o_ref[...] = acc + b_ref[...]


def _conv_tc(h_g, r_g, t_g, wh, wr, wt, bias_row, block_b):
    b = h_g.shape[0]
    grid = (b // block_b,)
    row_spec = pl.BlockSpec((block_b, D), lambda i: (i, 0))
    w_spec = pl.BlockSpec((D, NOUT), lambda i: (0, 0))
    return pl.pallas_call(
        _conv_body,
        grid=grid,
        in_specs=[row_spec, row_spec, row_spec, w_spec, w_spec, w_spec,
                  pl.BlockSpec((1, NOUT), lambda i: (0, 0))],
        out_specs=pl.BlockSpec((block_b, NOUT), lambda i: (i, 0)),
        out_shape=jax.ShapeDtypeStruct((b, NOUT), jnp.float32),
    )(h_g, r_g, t_g, wh, wr, wt, bias_row)


def _gather_sc(h_idx, r_idx, t_idx, ent_tab, rel_tab):
    b = h_idx.shape[0]
    info = plsc.get_sparse_core_info()
    nc = info.num_cores
    nw = nc * info.num_subcores
    rows_w = b // nw
    n_ch = rows_w // CH

    @functools.partial(
        pl.kernel,
        mesh=plsc.VectorSubcoreMesh(core_axis_name="c", subcore_axis_name="s"),
        compiler_params=pltpu.CompilerParams(use_tc_tiling_on_sc=False),
        out_type=(
            jax.ShapeDtypeStruct((b, D), jnp.float32),
            jax.ShapeDtypeStruct((b, D), jnp.float32),
            jax.ShapeDtypeStruct((b, D), jnp.float32),
        ),
        scratch_types=[
            pltpu.VMEM((rows_w,), jnp.int32),
            pltpu.VMEM((rows_w,), jnp.int32),
            pltpu.VMEM((rows_w,), jnp.int32),
            pltpu.VMEM((rows_w, D), jnp.float32),
            pltpu.VMEM((rows_w, D), jnp.float32),
            pltpu.VMEM((rows_w, D), jnp.float32),
            pltpu.SemaphoreType.DMA,
        ],
    )
    def gather_kernel(hi_hbm, ri_hbm, ti_hbm, ent_hbm, rel_hbm,
                      ho_hbm, ro_hbm, to_hbm,
                      hv, rv, tv, hb, rb, tb, sem):
        wid = lax.axis_index("s") * nc + lax.axis_index("c")
        base = wid * rows_w
        pltpu.sync_copy(hi_hbm.at[pl.ds(base, rows_w)], hv)
        pltpu.sync_copy(ri_hbm.at[pl.ds(base, rows_w)], rv)
        pltpu.sync_copy(ti_hbm.at[pl.ds(base, rows_w)], tv)
        handles = []
        for c in range(n_ch):
            sl = pl.ds(c * CH, CH)
            handles.append(pltpu.async_copy(ent_hbm.at[hv.at[sl]], hb.at[sl], sem))
            handles.append(pltpu.async_copy(rel_hbm.at[rv.at[sl]], rb.at[sl], sem))
            handles.append(pltpu.async_copy(ent_hbm.at[tv.at[sl]], tb.at[sl], sem))
        for hnd in handles:
            hnd.wait()
        out_sl = pl.ds(base, rows_w)
        pltpu.sync_copy(hb, ho_hbm.at[out_sl])
        pltpu.sync_copy(rb, ro_hbm.at[out_sl])
        pltpu.sync_copy(tb, to_hbm.at[out_sl])

    return gather_kernel(h_idx, r_idx, t_idx, ent_tab, rel_tab)


def kernel(inputs, entity_embeddings, relation_embeddings, conv_kernel, conv_bias):
    idx = inputs.astype(jnp.int32)
    h_g, r_g, t_g = _gather_sc(idx[:, 0], idx[:, 1], idx[:, 2],
                               entity_embeddings, relation_embeddings)
    w = _build_band_weights(conv_kernel)
    bias_row = jnp.tile(conv_bias, HOUT)[None, :]
    return _conv_tc(h_g, r_g, t_g, w[0], w[1], w[2], bias_row, 1024)
